# baseline (speedup 1.0000x reference)
import functools

import numpy as np
import jax
import jax.numpy as jnp
from jax import lax
from jax.experimental import pallas as pl
from jax.experimental.pallas import tpu as pltpu
from jax.experimental.pallas import tpu_sc as plsc

_B = 524288
_NW = 32
_PTS = _B // _NW
_CH = 1024
_NCHUNK = _PTS // _CH
_PAD = 8
_L = 16
_NVEC = _CH // _L
_MASK = (1 << 19) - 1
_P1 = int(np.uint32(2654435761).view(np.int32))
_P2 = int(np.uint32(805459861).view(np.int32))
_HOFF0 = 299008
_NLEV = 16
_LIN = ((16, 0), (32, 4096), (64, 36864))

_mesh = plsc.VectorSubcoreMesh(core_axis_name="c", subcore_axis_name="s")


def _phase1_hashed(xyz, idx8, f3, scale, resm1, offs):
    def body(i, _):
        s = i * _L
        x = xyz[0, pl.ds(s, _L)]
        y = xyz[1, pl.ds(s, _L)]
        z = xyz[2, pl.ds(s, _L)]
        px, py, pz = x * scale, y * scale, z * scale
        cx0 = px.astype(jnp.int32)
        cy0 = py.astype(jnp.int32)
        cz0 = pz.astype(jnp.int32)
        f3[pl.ds(s, _L)] = px - cx0.astype(jnp.float32)
        f3[pl.ds(_CH + s, _L)] = py - cy0.astype(jnp.float32)
        f3[pl.ds(2 * _CH + s, _L)] = pz - cz0.astype(jnp.float32)
        cx1 = jnp.minimum(cx0 + 1, resm1)
        cy1 = jnp.minimum(cy0 + 1, resm1)
        cz1 = jnp.minimum(cz0 + 1, resm1)
        hy0, hy1 = cy0 * _P1, cy1 * _P1
        hz0, hz1 = cz0 * _P2, cz1 * _P2
        hxy = ((cx0 ^ hy0, cx1 ^ hy0), (cx0 ^ hy1, cx1 ^ hy1))
        hz = (hz0, hz1)
        for c in range(8):
            bx, by, bz = c & 1, (c >> 1) & 1, c >> 2
            idx8[c, pl.ds(s, _L)] = ((hxy[by][bx] ^ hz[bz]) & _MASK) + offs
        return 0

    lax.fori_loop(0, _NVEC, body, 0, unroll=False)


def _phase1_linear(xyz, idx8, f3, res, offs):
    scale = float(res - 1)
    resm1 = res - 1
    s1, s2 = res, res * res

    def body(i, _):
        s = i * _L
        x = xyz[0, pl.ds(s, _L)]
        y = xyz[1, pl.ds(s, _L)]
        z = xyz[2, pl.ds(s, _L)]
        px, py, pz = x * scale, y * scale, z * scale
        cx0 = px.astype(jnp.int32)
        cy0 = py.astype(jnp.int32)
        cz0 = pz.astype(jnp.int32)
        f3[pl.ds(s, _L)] = px - cx0.astype(jnp.float32)
        f3[pl.ds(_CH + s, _L)] = py - cy0.astype(jnp.float32)
        f3[pl.ds(2 * _CH + s, _L)] = pz - cz0.astype(jnp.float32)
        cx1 = jnp.minimum(cx0 + 1, resm1)
        cy1 = jnp.minimum(cy0 + 1, resm1)
        cz1 = jnp.minimum(cz0 + 1, resm1)
        sxy = ((cx0 + cy0 * s1, cx1 + cy0 * s1), (cx0 + cy1 * s1, cx1 + cy1 * s1))
        sz = (cz0 * s2 + offs, cz1 * s2 + offs)
        for c in range(8):
            bx, by, bz = c & 1, (c >> 1) & 1, c >> 2
            idx8[c, pl.ds(s, _L)] = sxy[by][bx] + sz[bz]
        return 0

    lax.fori_loop(0, _NVEC, body, 0, unroll=False)


def _gather_corners(emb, idx8, vals8, sem):
    copies = []
    for c in range(8):
        copies.append(pltpu.async_copy(emb.at[idx8.at[c]], vals8.at[c], sem))
    for cp in copies:
        cp.wait()


def _phase2_accumulate(f3, vals8, outc, lev2):
    iota = lax.iota(jnp.int32, _L)
    zv = jnp.zeros((_L,), jnp.int32)
    ov = jnp.full((_L,), 1, jnp.int32)

    def body(i, _):
        s = i * _L
        p = iota + s
        fx = f3[pl.ds(s, _L)]
        fy = f3[pl.ds(_CH + s, _L)]
        fz = f3[pl.ds(2 * _CH + s, _L)]
        gx, gy, gz = 1.0 - fx, 1.0 - fy, 1.0 - fz
        v = [(plsc.load_gather(vals8.at[c], [p, zv]),
              plsc.load_gather(vals8.at[c], [p, ov])) for c in range(8)]
        pcol = p * (2 * _NLEV) + lev2
        for ch in (0, 1):
            u00 = v[0][ch] * gx + v[1][ch] * fx
            u10 = v[2][ch] * gx + v[3][ch] * fx
            u01 = v[4][ch] * gx + v[5][ch] * fx
            u11 = v[6][ch] * gx + v[7][ch] * fx
            m0 = u00 * gy + u10 * fy
            m1 = u01 * gy + u11 * fy
            plsc.store_scatter(outc, [pcol + ch], m0 * gz + m1 * fz)
        return 0

    lax.fori_loop(0, _NVEC, body, 0, unroll=False)


@functools.partial(
    pl.kernel,
    out_type=jax.ShapeDtypeStruct((_B * 2 * _NLEV,), jnp.float32),
    mesh=_mesh,
    scratch_types=[
        pltpu.VMEM((3, _CH), jnp.float32),
        pltpu.VMEM((8, _CH), jnp.int32),
        pltpu.VMEM((3 * _CH,), jnp.float32),
        pltpu.VMEM((8, _CH, _PAD), jnp.float32),
        pltpu.VMEM((_CH * 2 * _NLEV,), jnp.float32),
        pltpu.SemaphoreType.DMA,
    ],
    compiler_params=pltpu.CompilerParams(
        needs_layout_passes=False, use_tc_tiling_on_sc=False
    ),
)
def _hash_encode(inp_t, emb, out, xyz, idx8, f3, vals8, outc, sem):
    wid = lax.axis_index("s") * 2 + lax.axis_index("c")

    def chunk_body(k, _):
        base = wid * _PTS + k * _CH
        pltpu.sync_copy(inp_t.at[:, pl.ds(base, _CH)], xyz)

        for lev, (res, offs) in enumerate(_LIN):
            _phase1_linear(xyz, idx8, f3, res, offs)
            _gather_corners(emb, idx8, vals8, sem)
            _phase2_accumulate(f3, vals8, outc, 2 * lev)

        def lev_body(l, _):
            resm1 = (16 << l) - 1
            scale = resm1.astype(jnp.float32)
            offs = _HOFF0 + ((l - 3) << 19)
            _phase1_hashed(xyz, idx8, f3, scale, resm1, offs)
            _gather_corners(emb, idx8, vals8, sem)
            _phase2_accumulate(f3, vals8, outc, 2 * l)
            return 0

        lax.fori_loop(3, _NLEV, lev_body, 0, unroll=False)
        pltpu.sync_copy(outc, out.at[pl.ds(base * 2 * _NLEV, _CH * 2 * _NLEV)])
        return 0

    lax.fori_loop(0, _NCHUNK, chunk_body, 0, unroll=False)


def kernel(inputs, embeddings):
    inp_t = inputs.T
    emb8 = jnp.pad(embeddings, ((0, 0), (0, _PAD - 2)))
    out = _hash_encode(jnp.asarray(inp_t), emb8)
    return out.reshape(_B, 2 * _NLEV)


# speedup vs baseline: 1.1628x; 1.1628x over previous
import functools

import numpy as np
import jax
import jax.numpy as jnp
from jax import lax
from jax.experimental import pallas as pl
from jax.experimental.pallas import tpu as pltpu
from jax.experimental.pallas import tpu_sc as plsc

_B = 524288
_NW = 32
_PTS = _B // _NW
_CH = 1024
_NCHUNK = _PTS // _CH
_PAD = 8
_L = 16
_NVEC = _CH // _L
_MASK = (1 << 19) - 1
_P1 = int(np.uint32(2654435761).view(np.int32))
_P2 = int(np.uint32(805459861).view(np.int32))
_HOFF0 = 299008
_NLEV = 16
_LIN = ((16, 0), (32, 4096), (64, 36864))

_mesh = plsc.VectorSubcoreMesh(core_axis_name="c", subcore_axis_name="s")


def _phase1_hashed(xyz, idx8, sub8, f3, scale, resm1, offs):
    def body(i, _):
        s = i * _L
        x = xyz[0, pl.ds(s, _L)]
        y = xyz[1, pl.ds(s, _L)]
        z = xyz[2, pl.ds(s, _L)]
        px, py, pz = x * scale, y * scale, z * scale
        cx0 = px.astype(jnp.int32)
        cy0 = py.astype(jnp.int32)
        cz0 = pz.astype(jnp.int32)
        f3[pl.ds(s, _L)] = px - cx0.astype(jnp.float32)
        f3[pl.ds(_CH + s, _L)] = py - cy0.astype(jnp.float32)
        f3[pl.ds(2 * _CH + s, _L)] = pz - cz0.astype(jnp.float32)
        cx1 = jnp.minimum(cx0 + 1, resm1)
        cy1 = jnp.minimum(cy0 + 1, resm1)
        cz1 = jnp.minimum(cz0 + 1, resm1)
        hy0, hy1 = cy0 * _P1, cy1 * _P1
        hz0, hz1 = cz0 * _P2, cz1 * _P2
        hxy = ((cx0 ^ hy0, cx1 ^ hy0), (cx0 ^ hy1, cx1 ^ hy1))
        hz = (hz0, hz1)
        for c in range(8):
            bx, by, bz = c & 1, (c >> 1) & 1, c >> 2
            idx = ((hxy[by][bx] ^ hz[bz]) & _MASK) + offs
            idx8[c, pl.ds(s, _L)] = idx >> 2
            sub8[c, pl.ds(s, _L)] = (idx & 3) << 1
        return 0

    lax.fori_loop(0, _NVEC, body, 0, unroll=False)


def _phase1_linear(xyz, idx8, sub8, f3, res, offs):
    scale = float(res - 1)
    resm1 = res - 1
    s1, s2 = res, res * res

    def body(i, _):
        s = i * _L
        x = xyz[0, pl.ds(s, _L)]
        y = xyz[1, pl.ds(s, _L)]
        z = xyz[2, pl.ds(s, _L)]
        px, py, pz = x * scale, y * scale, z * scale
        cx0 = px.astype(jnp.int32)
        cy0 = py.astype(jnp.int32)
        cz0 = pz.astype(jnp.int32)
        f3[pl.ds(s, _L)] = px - cx0.astype(jnp.float32)
        f3[pl.ds(_CH + s, _L)] = py - cy0.astype(jnp.float32)
        f3[pl.ds(2 * _CH + s, _L)] = pz - cz0.astype(jnp.float32)
        cx1 = jnp.minimum(cx0 + 1, resm1)
        cy1 = jnp.minimum(cy0 + 1, resm1)
        cz1 = jnp.minimum(cz0 + 1, resm1)
        sxy = ((cx0 + cy0 * s1, cx1 + cy0 * s1), (cx0 + cy1 * s1, cx1 + cy1 * s1))
        sz = (cz0 * s2 + offs, cz1 * s2 + offs)
        for c in range(8):
            bx, by, bz = c & 1, (c >> 1) & 1, c >> 2
            idx = sxy[by][bx] + sz[bz]
            idx8[c, pl.ds(s, _L)] = idx >> 2
            sub8[c, pl.ds(s, _L)] = (idx & 3) << 1
        return 0

    lax.fori_loop(0, _NVEC, body, 0, unroll=False)


def _gather_corners(emb, idx8, vals8, sem):
    copies = []
    for c in range(8):
        copies.append(pltpu.async_copy(emb.at[idx8.at[c]], vals8.at[c], sem))
    for cp in copies:
        cp.wait()


def _phase2_accumulate(f3, sub8, vals8, outc, lev2):
    iota = lax.iota(jnp.int32, _L)

    def body(i, _):
        s = i * _L
        p = iota + s
        fx = f3[pl.ds(s, _L)]
        fy = f3[pl.ds(_CH + s, _L)]
        fz = f3[pl.ds(2 * _CH + s, _L)]
        gx, gy, gz = 1.0 - fx, 1.0 - fy, 1.0 - fz
        subs = [sub8[c, pl.ds(s, _L)] for c in range(8)]
        v = [(plsc.load_gather(vals8.at[c], [p, subs[c]]),
              plsc.load_gather(vals8.at[c], [p, subs[c] + 1])) for c in range(8)]
        pcol = p * (2 * _NLEV) + lev2
        for ch in (0, 1):
            u00 = v[0][ch] * gx + v[1][ch] * fx
            u10 = v[2][ch] * gx + v[3][ch] * fx
            u01 = v[4][ch] * gx + v[5][ch] * fx
            u11 = v[6][ch] * gx + v[7][ch] * fx
            m0 = u00 * gy + u10 * fy
            m1 = u01 * gy + u11 * fy
            plsc.store_scatter(outc, [pcol + ch], m0 * gz + m1 * fz)
        return 0

    lax.fori_loop(0, _NVEC, body, 0, unroll=False)


@functools.partial(
    pl.kernel,
    out_type=jax.ShapeDtypeStruct((_B * 2 * _NLEV,), jnp.float32),
    mesh=_mesh,
    scratch_types=[
        pltpu.VMEM((3, _CH), jnp.float32),
        pltpu.VMEM((8, _CH), jnp.int32),
        pltpu.VMEM((8, _CH), jnp.int32),
        pltpu.VMEM((3 * _CH,), jnp.float32),
        pltpu.VMEM((8, _CH, _PAD), jnp.float32),
        pltpu.VMEM((_CH * 2 * _NLEV,), jnp.float32),
        pltpu.SemaphoreType.DMA,
    ],
    compiler_params=pltpu.CompilerParams(
        needs_layout_passes=False, use_tc_tiling_on_sc=False
    ),
)
def _hash_encode(inp_t, emb, out, xyz, idx8, sub8, f3, vals8, outc, sem):
    wid = lax.axis_index("s") * 2 + lax.axis_index("c")

    def chunk_body(k, _):
        base = wid * _PTS + k * _CH
        pltpu.sync_copy(inp_t.at[:, pl.ds(base, _CH)], xyz)

        for lev, (res, offs) in enumerate(_LIN):
            _phase1_linear(xyz, idx8, sub8, f3, res, offs)
            _gather_corners(emb, idx8, vals8, sem)
            _phase2_accumulate(f3, sub8, vals8, outc, 2 * lev)

        def lev_body(l, _):
            resm1 = (16 << l) - 1
            scale = resm1.astype(jnp.float32)
            offs = _HOFF0 + ((l - 3) << 19)
            _phase1_hashed(xyz, idx8, sub8, f3, scale, resm1, offs)
            _gather_corners(emb, idx8, vals8, sem)
            _phase2_accumulate(f3, sub8, vals8, outc, 2 * l)
            return 0

        lax.fori_loop(3, _NLEV, lev_body, 0, unroll=False)
        pltpu.sync_copy(outc, out.at[pl.ds(base * 2 * _NLEV, _CH * 2 * _NLEV)])
        return 0

    lax.fori_loop(0, _NCHUNK, chunk_body, 0, unroll=False)


def kernel(inputs, embeddings):
    inp_t = inputs.T
    embq = embeddings.reshape(-1, _PAD)
    out = _hash_encode(jnp.asarray(inp_t), embq)
    return out.reshape(_B, 2 * _NLEV)


# speedup vs baseline: 3.4733x; 2.9869x over previous
import functools

import numpy as np
import jax
import jax.numpy as jnp
from jax import lax
from jax.experimental import pallas as pl
from jax.experimental.pallas import tpu as pltpu
from jax.experimental.pallas import tpu_sc as plsc

_B = 524288
_NW = 32
_PTS = _B // _NW
_CH = 1024
_NCHUNK = _PTS // _CH
_PAD = 8
_L = 16
_NVEC = _CH // _L
_MASK = (1 << 19) - 1
_P1 = int(np.uint32(2654435761).view(np.int32))
_P2 = int(np.uint32(805459861).view(np.int32))
_HOFF0 = 299008
_NLEV = 16
_TOTAL_PARAMS = 7114752
_LIN = ((16, 0), (32, 4096), (64, 36864))

_mesh = plsc.VectorSubcoreMesh(core_axis_name="c", subcore_axis_name="s")
_cparams = pltpu.CompilerParams(
    needs_layout_passes=False, use_tc_tiling_on_sc=False
)

_TOTF = _TOTAL_PARAMS * 2
_RPW = _TOTF // _NW
_RNB = 9
_RITER = _RPW // (256 * _RNB)


@functools.partial(
    pl.kernel,
    out_type=jax.ShapeDtypeStruct((_TOTF,), jnp.float32),
    mesh=_mesh,
    scratch_types=[
        pltpu.VMEM((_RNB * 256,), jnp.float32),
        pltpu.VMEM((_RNB * 256,), jnp.float32),
        pltpu.SemaphoreType.DMA,
    ],
    compiler_params=_cparams,
)
def _relayout(embv, out, inbuf, outbuf, sem):
    wid = lax.axis_index("s") * 2 + lax.axis_index("c")
    base = wid * _RPW
    iota = lax.iota(jnp.int32, _L)
    pat = ((iota & 1) << 7) + (iota >> 1)

    def body(t, _):
        off = base + t * (_RNB * 256)
        pltpu.sync_copy(embv.at[pl.ds(off, _RNB * 256)], inbuf)

        def blk(b, _b):
            pb = pat + b * 256

            for g in range(16):
                outbuf[pl.ds(b * 256 + 16 * g, _L)] = plsc.load_gather(
                    inbuf, [pb + 8 * g])
            return 0

        lax.fori_loop(0, _RNB, blk, 0, unroll=False)
        pltpu.sync_copy(outbuf, out.at[pl.ds(off, _RNB * 256)])
        return 0

    lax.fori_loop(0, _RITER, body, 0, unroll=False)


def _phase1_hashed(xyz, idx8, sub8, f3, scale, resm1, offs):
    def body(i, _):
        s = i * _L
        x = xyz[0, pl.ds(s, _L)]
        y = xyz[1, pl.ds(s, _L)]
        z = xyz[2, pl.ds(s, _L)]
        px, py, pz = x * scale, y * scale, z * scale
        cx0 = px.astype(jnp.int32)
        cy0 = py.astype(jnp.int32)
        cz0 = pz.astype(jnp.int32)
        f3[pl.ds(s, _L)] = px - cx0.astype(jnp.float32)
        f3[pl.ds(_CH + s, _L)] = py - cy0.astype(jnp.float32)
        f3[pl.ds(2 * _CH + s, _L)] = pz - cz0.astype(jnp.float32)
        cx1 = jnp.minimum(cx0 + 1, resm1)
        cy1 = jnp.minimum(cy0 + 1, resm1)
        cz1 = jnp.minimum(cz0 + 1, resm1)
        hy0, hy1 = cy0 * _P1, cy1 * _P1
        hz0, hz1 = cz0 * _P2, cz1 * _P2
        hxy = ((cx0 ^ hy0, cx1 ^ hy0), (cx0 ^ hy1, cx1 ^ hy1))
        hz = (hz0, hz1)
        for c in range(8):
            bx, by, bz = c & 1, (c >> 1) & 1, c >> 2
            idx = ((hxy[by][bx] ^ hz[bz]) & _MASK) + offs
            idx8[c, pl.ds(s, _L)] = idx >> 2
            sub8[c, pl.ds(s, _L)] = (idx & 3) << 1
        return 0

    lax.fori_loop(0, _NVEC, body, 0, unroll=False)


def _phase1_linear(xyz, idx8, sub8, f3, res, offs):
    scale = float(res - 1)
    resm1 = res - 1
    s1, s2 = res, res * res

    def body(i, _):
        s = i * _L
        x = xyz[0, pl.ds(s, _L)]
        y = xyz[1, pl.ds(s, _L)]
        z = xyz[2, pl.ds(s, _L)]
        px, py, pz = x * scale, y * scale, z * scale
        cx0 = px.astype(jnp.int32)
        cy0 = py.astype(jnp.int32)
        cz0 = pz.astype(jnp.int32)
        f3[pl.ds(s, _L)] = px - cx0.astype(jnp.float32)
        f3[pl.ds(_CH + s, _L)] = py - cy0.astype(jnp.float32)
        f3[pl.ds(2 * _CH + s, _L)] = pz - cz0.astype(jnp.float32)
        cx1 = jnp.minimum(cx0 + 1, resm1)
        cy1 = jnp.minimum(cy0 + 1, resm1)
        cz1 = jnp.minimum(cz0 + 1, resm1)
        sxy = ((cx0 + cy0 * s1, cx1 + cy0 * s1), (cx0 + cy1 * s1, cx1 + cy1 * s1))
        sz = (cz0 * s2 + offs, cz1 * s2 + offs)
        for c in range(8):
            bx, by, bz = c & 1, (c >> 1) & 1, c >> 2
            idx = sxy[by][bx] + sz[bz]
            idx8[c, pl.ds(s, _L)] = idx >> 2
            sub8[c, pl.ds(s, _L)] = (idx & 3) << 1
        return 0

    lax.fori_loop(0, _NVEC, body, 0, unroll=False)


def _gather_corners(emb, idx8, vals8, sem):
    copies = []
    for c in range(8):
        copies.append(pltpu.async_copy(emb.at[idx8.at[c]], vals8.at[c], sem))
    for cp in copies:
        cp.wait()


def _phase2_accumulate(f3, sub8, vals8, outc, lev2):
    iota = lax.iota(jnp.int32, _L)

    def body(i, _):
        s = i * _L
        p = iota + s
        fx = f3[pl.ds(s, _L)]
        fy = f3[pl.ds(_CH + s, _L)]
        fz = f3[pl.ds(2 * _CH + s, _L)]
        gx, gy, gz = 1.0 - fx, 1.0 - fy, 1.0 - fz
        subs = [sub8[c, pl.ds(s, _L)] for c in range(8)]
        v = [(plsc.load_gather(vals8.at[c], [p, subs[c]]),
              plsc.load_gather(vals8.at[c], [p, subs[c] + 1])) for c in range(8)]
        pcol = p * (2 * _NLEV) + lev2
        for ch in (0, 1):
            u00 = v[0][ch] * gx + v[1][ch] * fx
            u10 = v[2][ch] * gx + v[3][ch] * fx
            u01 = v[4][ch] * gx + v[5][ch] * fx
            u11 = v[6][ch] * gx + v[7][ch] * fx
            m0 = u00 * gy + u10 * fy
            m1 = u01 * gy + u11 * fy
            plsc.store_scatter(outc, [pcol + ch], m0 * gz + m1 * fz)
        return 0

    lax.fori_loop(0, _NVEC, body, 0, unroll=False)


@functools.partial(
    pl.kernel,
    out_type=jax.ShapeDtypeStruct((_B * 2 * _NLEV,), jnp.float32),
    mesh=_mesh,
    scratch_types=[
        pltpu.VMEM((3, _CH), jnp.float32),
        pltpu.VMEM((8, _CH), jnp.int32),
        pltpu.VMEM((8, _CH), jnp.int32),
        pltpu.VMEM((3 * _CH,), jnp.float32),
        pltpu.VMEM((8, _CH, _PAD), jnp.float32),
        pltpu.VMEM((_CH * 2 * _NLEV,), jnp.float32),
        pltpu.SemaphoreType.DMA,
    ],
    compiler_params=_cparams,
)
def _hash_encode(inp_t, emb, out, xyz, idx8, sub8, f3, vals8, outc, sem):
    wid = lax.axis_index("s") * 2 + lax.axis_index("c")

    def chunk_body(k, _):
        base = wid * _PTS + k * _CH
        pltpu.sync_copy(inp_t.at[:, pl.ds(base, _CH)], xyz)

        for lev, (res, offs) in enumerate(_LIN):
            _phase1_linear(xyz, idx8, sub8, f3, res, offs)
            _gather_corners(emb, idx8, vals8, sem)
            _phase2_accumulate(f3, sub8, vals8, outc, 2 * lev)

        def lev_body(l, _):
            resm1 = (16 << l) - 1
            scale = resm1.astype(jnp.float32)
            offs = _HOFF0 + ((l - 3) << 19)
            _phase1_hashed(xyz, idx8, sub8, f3, scale, resm1, offs)
            _gather_corners(emb, idx8, vals8, sem)
            _phase2_accumulate(f3, sub8, vals8, outc, 2 * l)
            return 0

        lax.fori_loop(3, _NLEV, lev_body, 0, unroll=False)
        pltpu.sync_copy(outc, out.at[pl.ds(base * 2 * _NLEV, _CH * 2 * _NLEV)])
        return 0

    lax.fori_loop(0, _NCHUNK, chunk_body, 0, unroll=False)


def kernel(inputs, embeddings):
    inp_t = inputs.T
    embv = embeddings.reshape(-1, 128, 2).transpose(0, 2, 1).reshape(-1)
    embq = _relayout(embv).reshape(-1, _PAD)
    out = _hash_encode(jnp.asarray(inp_t), embq)
    return out.reshape(_B, 2 * _NLEV)


# speedup vs baseline: 4.8602x; 1.3993x over previous
import functools

import numpy as np
import jax
import jax.numpy as jnp
from jax import lax
from jax.experimental import pallas as pl
from jax.experimental.pallas import tpu as pltpu
from jax.experimental.pallas import tpu_sc as plsc

_B = 524288
_NW = 32
_PTS = _B // _NW
_CH = 512
_NCHUNK = _PTS // _CH
_PAD = 8
_L = 16
_NVEC = _CH // _L
_MASK = (1 << 19) - 1
_P1 = int(np.uint32(2654435761).view(np.int32))
_P2 = int(np.uint32(805459861).view(np.int32))
_HOFF0 = 299008
_NLEV = 16
_TOTAL_PARAMS = 7114752
_LIN = ((16, 0), (32, 4096), (64, 36864))

_mesh = plsc.VectorSubcoreMesh(core_axis_name="c", subcore_axis_name="s")
_cparams = pltpu.CompilerParams(
    needs_layout_passes=False, use_tc_tiling_on_sc=False
)

_TOTF = _TOTAL_PARAMS * 2
_RPW = _TOTF // _NW
_RNB = 9
_RITER = _RPW // (256 * _RNB)


@functools.partial(
    pl.kernel,
    out_type=jax.ShapeDtypeStruct((_TOTF,), jnp.float32),
    mesh=_mesh,
    scratch_types=[
        pltpu.VMEM((_RNB * 256,), jnp.float32),
        pltpu.VMEM((_RNB * 256,), jnp.float32),
        pltpu.SemaphoreType.DMA,
    ],
    compiler_params=_cparams,
)
def _relayout(embv, out, inbuf, outbuf, sem):
    wid = lax.axis_index("s") * 2 + lax.axis_index("c")
    base = wid * _RPW
    iota = lax.iota(jnp.int32, _L)
    pat = ((iota & 1) << 7) + (iota >> 1)

    def body(t, _):
        off = base + t * (_RNB * 256)
        pltpu.sync_copy(embv.at[pl.ds(off, _RNB * 256)], inbuf)

        def blk(b, _b):
            pb = pat + b * 256

            for g in range(16):
                outbuf[pl.ds(b * 256 + 16 * g, _L)] = plsc.load_gather(
                    inbuf, [pb + 8 * g])
            return 0

        lax.fori_loop(0, _RNB, blk, 0, unroll=False)
        pltpu.sync_copy(outbuf, out.at[pl.ds(off, _RNB * 256)])
        return 0

    lax.fori_loop(0, _RITER, body, 0, unroll=False)


def _phase1_hashed(xyz, idx8, sub8, f3, b, lev):
    resm1 = (16 << lev) - 1
    scale = float(resm1)
    offs = _HOFF0 + ((lev - 3) << 19)

    def body(i, _):
        s = i * _L
        x = xyz[0, pl.ds(s, _L)]
        y = xyz[1, pl.ds(s, _L)]
        z = xyz[2, pl.ds(s, _L)]
        px, py, pz = x * scale, y * scale, z * scale
        cx0 = px.astype(jnp.int32)
        cy0 = py.astype(jnp.int32)
        cz0 = pz.astype(jnp.int32)
        f3[b, pl.ds(s, _L)] = px - cx0.astype(jnp.float32)
        f3[b, pl.ds(_CH + s, _L)] = py - cy0.astype(jnp.float32)
        f3[b, pl.ds(2 * _CH + s, _L)] = pz - cz0.astype(jnp.float32)
        cx1 = jnp.minimum(cx0 + 1, resm1)
        cy1 = jnp.minimum(cy0 + 1, resm1)
        cz1 = jnp.minimum(cz0 + 1, resm1)
        hy0, hy1 = cy0 * _P1, cy1 * _P1
        hz0, hz1 = cz0 * _P2, cz1 * _P2
        hxy = ((cx0 ^ hy0, cx1 ^ hy0), (cx0 ^ hy1, cx1 ^ hy1))
        hz = (hz0, hz1)
        for c in range(8):
            bx, by, bz = c & 1, (c >> 1) & 1, c >> 2
            idx = ((hxy[by][bx] ^ hz[bz]) & _MASK) + offs
            idx8[b, c, pl.ds(s, _L)] = idx >> 2
            sub8[b, c, pl.ds(s, _L)] = (idx & 3) << 1
        return 0

    lax.fori_loop(0, _NVEC, body, 0, unroll=False)


def _phase1_linear(xyz, idx8, sub8, f3, b, res, offs):
    scale = float(res - 1)
    resm1 = res - 1
    s1, s2 = res, res * res

    def body(i, _):
        s = i * _L
        x = xyz[0, pl.ds(s, _L)]
        y = xyz[1, pl.ds(s, _L)]
        z = xyz[2, pl.ds(s, _L)]
        px, py, pz = x * scale, y * scale, z * scale
        cx0 = px.astype(jnp.int32)
        cy0 = py.astype(jnp.int32)
        cz0 = pz.astype(jnp.int32)
        f3[b, pl.ds(s, _L)] = px - cx0.astype(jnp.float32)
        f3[b, pl.ds(_CH + s, _L)] = py - cy0.astype(jnp.float32)
        f3[b, pl.ds(2 * _CH + s, _L)] = pz - cz0.astype(jnp.float32)
        cx1 = jnp.minimum(cx0 + 1, resm1)
        cy1 = jnp.minimum(cy0 + 1, resm1)
        cz1 = jnp.minimum(cz0 + 1, resm1)
        sxy = ((cx0 + cy0 * s1, cx1 + cy0 * s1), (cx0 + cy1 * s1, cx1 + cy1 * s1))
        sz = (cz0 * s2 + offs, cz1 * s2 + offs)
        for c in range(8):
            bx, by, bz = c & 1, (c >> 1) & 1, c >> 2
            idx = sxy[by][bx] + sz[bz]
            idx8[b, c, pl.ds(s, _L)] = idx >> 2
            sub8[b, c, pl.ds(s, _L)] = (idx & 3) << 1
        return 0

    lax.fori_loop(0, _NVEC, body, 0, unroll=False)


def _phase2_accumulate(f3, sub8, vals8, outc, b, lev2):
    iota = lax.iota(jnp.int32, _L)

    def body(i, _):
        s = i * _L
        p = iota + s
        fx = f3[b, pl.ds(s, _L)]
        fy = f3[b, pl.ds(_CH + s, _L)]
        fz = f3[b, pl.ds(2 * _CH + s, _L)]
        gx, gy, gz = 1.0 - fx, 1.0 - fy, 1.0 - fz
        subs = [sub8[b, c, pl.ds(s, _L)] for c in range(8)]
        v = [(plsc.load_gather(vals8.at[b, c], [p, subs[c]]),
              plsc.load_gather(vals8.at[b, c], [p, subs[c] + 1]))
             for c in range(8)]
        pcol = p * (2 * _NLEV) + lev2
        for ch in (0, 1):
            u00 = v[0][ch] * gx + v[1][ch] * fx
            u10 = v[2][ch] * gx + v[3][ch] * fx
            u01 = v[4][ch] * gx + v[5][ch] * fx
            u11 = v[6][ch] * gx + v[7][ch] * fx
            m0 = u00 * gy + u10 * fy
            m1 = u01 * gy + u11 * fy
            plsc.store_scatter(outc, [pcol + ch], m0 * gz + m1 * fz)
        return 0

    lax.fori_loop(0, _NVEC, body, 0, unroll=False)


@functools.partial(
    pl.kernel,
    out_type=jax.ShapeDtypeStruct((_B * 2 * _NLEV,), jnp.float32),
    mesh=_mesh,
    scratch_types=[
        pltpu.VMEM((3, _CH), jnp.float32),
        pltpu.VMEM((2, 8, _CH), jnp.int32),
        pltpu.VMEM((2, 8, _CH), jnp.int32),
        pltpu.VMEM((2, 3 * _CH), jnp.float32),
        pltpu.VMEM((2, 8, _CH, _PAD), jnp.float32),
        pltpu.VMEM((_CH * 2 * _NLEV,), jnp.float32),
        pltpu.SemaphoreType.DMA,
        pltpu.SemaphoreType.DMA,
    ],
    compiler_params=_cparams,
)
def _hash_encode(inp_t, emb, out, xyz, idx8, sub8, f3, vals8, outc, sem0, sem1):
    wid = lax.axis_index("s") * 2 + lax.axis_index("c")
    sems = (sem0, sem1)

    def chunk_body(k, _):
        base = wid * _PTS + k * _CH
        pltpu.sync_copy(inp_t.at[:, pl.ds(base, _CH)], xyz)
        pending = None
        for lev in range(_NLEV):
            b = lev & 1
            if lev < 3:
                res, offs = _LIN[lev]
                _phase1_linear(xyz, idx8, sub8, f3, b, res, offs)
            else:
                _phase1_hashed(xyz, idx8, sub8, f3, b, lev)
            cps = [pltpu.async_copy(emb.at[idx8.at[b, c]], vals8.at[b, c],
                                    sems[b]) for c in range(8)]
            if pending is not None:
                pb, pcps, plev = pending
                for cp in pcps:
                    cp.wait()
                _phase2_accumulate(f3, sub8, vals8, outc, pb, 2 * plev)
            pending = (b, cps, lev)
        pb, pcps, plev = pending
        for cp in pcps:
            cp.wait()
        _phase2_accumulate(f3, sub8, vals8, outc, pb, 2 * plev)
        pltpu.sync_copy(outc, out.at[pl.ds(base * 2 * _NLEV, _CH * 2 * _NLEV)])
        return 0

    lax.fori_loop(0, _NCHUNK, chunk_body, 0, unroll=False)


def kernel(inputs, embeddings):
    inp_t = inputs.T
    embv = embeddings.reshape(-1, 128, 2).transpose(0, 2, 1).reshape(-1)
    embq = _relayout(embv).reshape(-1, _PAD)
    out = _hash_encode(jnp.asarray(inp_t), embq)
    return out.reshape(_B, 2 * _NLEV)


# speedup vs baseline: 5.0579x; 1.0407x over previous
import functools

import numpy as np
import jax
import jax.numpy as jnp
from jax import lax
from jax.experimental import pallas as pl
from jax.experimental.pallas import tpu as pltpu
from jax.experimental.pallas import tpu_sc as plsc

_B = 524288
_NW = 32
_PTS = _B // _NW
_CH = 512
_NCHUNK = _PTS // _CH
_PAD = 8
_L = 16
_NVEC = _CH // _L
_MASK = (1 << 19) - 1
_P1 = int(np.uint32(2654435761).view(np.int32))
_P2 = int(np.uint32(805459861).view(np.int32))
_HOFF0 = 299008
_NLEV = 16
_TOTAL_PARAMS = 7114752
_LIN = ((16, 0), (32, 4096), (64, 36864))

_mesh = plsc.VectorSubcoreMesh(core_axis_name="c", subcore_axis_name="s")
_cparams = pltpu.CompilerParams(
    needs_layout_passes=False, use_tc_tiling_on_sc=False
)

_TOTF = _TOTAL_PARAMS * 2
_RPW = _TOTF // _NW
_RNB = 193
_RITER = _RPW // (256 * _RNB)


@functools.partial(
    pl.kernel,
    out_type=jax.ShapeDtypeStruct((_TOTF,), jnp.float32),
    mesh=_mesh,
    scratch_types=[
        pltpu.VMEM((_RNB * 256,), jnp.float32),
        pltpu.VMEM((_RNB * 256,), jnp.float32),
        pltpu.SemaphoreType.DMA,
    ],
    compiler_params=_cparams,
)
def _relayout(embv, out, inbuf, outbuf, sem):
    wid = lax.axis_index("s") * 2 + lax.axis_index("c")
    base = wid * _RPW
    iota = lax.iota(jnp.int32, _L)
    pat = ((iota & 1) << 7) + (iota >> 1)

    def body(t, _):
        off = base + t * (_RNB * 256)
        pltpu.sync_copy(embv.at[pl.ds(off, _RNB * 256)], inbuf)

        def blk(b, _b):
            pb = pat + b * 256

            for g in range(16):
                outbuf[pl.ds(b * 256 + 16 * g, _L)] = plsc.load_gather(
                    inbuf, [pb + 8 * g])
            return 0

        lax.fori_loop(0, _RNB, blk, 0, unroll=False)
        pltpu.sync_copy(outbuf, out.at[pl.ds(off, _RNB * 256)])
        return 0

    lax.fori_loop(0, _RITER, body, 0, unroll=False)


def _phase1_hashed(xyz, idx8, sub8, f3, b, lev):
    resm1 = (16 << lev) - 1
    scale = float(resm1)
    offs = _HOFF0 + ((lev - 3) << 19)

    def body(i, _):
        s = i * _L
        x = xyz[0, pl.ds(s, _L)]
        y = xyz[1, pl.ds(s, _L)]
        z = xyz[2, pl.ds(s, _L)]
        px, py, pz = x * scale, y * scale, z * scale
        cx0 = px.astype(jnp.int32)
        cy0 = py.astype(jnp.int32)
        cz0 = pz.astype(jnp.int32)
        f3[b, pl.ds(s, _L)] = px - cx0.astype(jnp.float32)
        f3[b, pl.ds(_CH + s, _L)] = py - cy0.astype(jnp.float32)
        f3[b, pl.ds(2 * _CH + s, _L)] = pz - cz0.astype(jnp.float32)
        cx1 = jnp.minimum(cx0 + 1, resm1)
        cy1 = jnp.minimum(cy0 + 1, resm1)
        cz1 = jnp.minimum(cz0 + 1, resm1)
        hy0, hy1 = cy0 * _P1, cy1 * _P1
        hz0, hz1 = cz0 * _P2, cz1 * _P2
        hxy = ((cx0 ^ hy0, cx1 ^ hy0), (cx0 ^ hy1, cx1 ^ hy1))
        hz = (hz0, hz1)
        for c in range(8):
            bx, by, bz = c & 1, (c >> 1) & 1, c >> 2
            idx = ((hxy[by][bx] ^ hz[bz]) & _MASK) + offs
            idx8[b, c, pl.ds(s, _L)] = idx >> 2
            sub8[b, c, pl.ds(s, _L)] = (idx & 3) << 1
        return 0

    lax.fori_loop(0, _NVEC, body, 0, unroll=False)


def _phase1_linear(xyz, idx8, sub8, f3, b, res, offs):
    scale = float(res - 1)
    resm1 = res - 1
    s1, s2 = res, res * res

    def body(i, _):
        s = i * _L
        x = xyz[0, pl.ds(s, _L)]
        y = xyz[1, pl.ds(s, _L)]
        z = xyz[2, pl.ds(s, _L)]
        px, py, pz = x * scale, y * scale, z * scale
        cx0 = px.astype(jnp.int32)
        cy0 = py.astype(jnp.int32)
        cz0 = pz.astype(jnp.int32)
        f3[b, pl.ds(s, _L)] = px - cx0.astype(jnp.float32)
        f3[b, pl.ds(_CH + s, _L)] = py - cy0.astype(jnp.float32)
        f3[b, pl.ds(2 * _CH + s, _L)] = pz - cz0.astype(jnp.float32)
        cx1 = jnp.minimum(cx0 + 1, resm1)
        cy1 = jnp.minimum(cy0 + 1, resm1)
        cz1 = jnp.minimum(cz0 + 1, resm1)
        sxy = ((cx0 + cy0 * s1, cx1 + cy0 * s1), (cx0 + cy1 * s1, cx1 + cy1 * s1))
        sz = (cz0 * s2 + offs, cz1 * s2 + offs)
        for c in range(8):
            bx, by, bz = c & 1, (c >> 1) & 1, c >> 2
            idx = sxy[by][bx] + sz[bz]
            idx8[b, c, pl.ds(s, _L)] = idx >> 2
            sub8[b, c, pl.ds(s, _L)] = (idx & 3) << 1
        return 0

    lax.fori_loop(0, _NVEC, body, 0, unroll=False)


def _phase2_accumulate(f3, sub8, vals8, outc, b, lev2):
    iota = lax.iota(jnp.int32, _L)

    def body(i, _):
        s = i * _L
        p = iota + s
        fx = f3[b, pl.ds(s, _L)]
        fy = f3[b, pl.ds(_CH + s, _L)]
        fz = f3[b, pl.ds(2 * _CH + s, _L)]
        gx, gy, gz = 1.0 - fx, 1.0 - fy, 1.0 - fz
        subs = [sub8[b, c, pl.ds(s, _L)] for c in range(8)]
        v = [(plsc.load_gather(vals8.at[b, c], [p, subs[c]]),
              plsc.load_gather(vals8.at[b, c], [p, subs[c] + 1]))
             for c in range(8)]
        pcol = p * (2 * _NLEV) + lev2
        for ch in (0, 1):
            u00 = v[0][ch] * gx + v[1][ch] * fx
            u10 = v[2][ch] * gx + v[3][ch] * fx
            u01 = v[4][ch] * gx + v[5][ch] * fx
            u11 = v[6][ch] * gx + v[7][ch] * fx
            m0 = u00 * gy + u10 * fy
            m1 = u01 * gy + u11 * fy
            plsc.store_scatter(outc, [pcol + ch], m0 * gz + m1 * fz)
        return 0

    lax.fori_loop(0, _NVEC, body, 0, unroll=False)


@functools.partial(
    pl.kernel,
    out_type=jax.ShapeDtypeStruct((_B * 2 * _NLEV,), jnp.float32),
    mesh=_mesh,
    scratch_types=[
        pltpu.VMEM((3, _CH), jnp.float32),
        pltpu.VMEM((2, 8, _CH), jnp.int32),
        pltpu.VMEM((2, 8, _CH), jnp.int32),
        pltpu.VMEM((2, 3 * _CH), jnp.float32),
        pltpu.VMEM((2, 8, _CH, _PAD), jnp.float32),
        pltpu.VMEM((_CH * 2 * _NLEV,), jnp.float32),
        pltpu.SemaphoreType.DMA,
        pltpu.SemaphoreType.DMA,
    ],
    compiler_params=_cparams,
)
def _hash_encode(inp_t, emb, out, xyz, idx8, sub8, f3, vals8, outc, sem0, sem1):
    wid = lax.axis_index("s") * 2 + lax.axis_index("c")
    sems = (sem0, sem1)

    def chunk_body(k, _):
        base = wid * _PTS + k * _CH
        pltpu.sync_copy(inp_t.at[:, pl.ds(base, _CH)], xyz)
        pending = None
        for lev in range(_NLEV):
            b = lev & 1
            if lev < 3:
                res, offs = _LIN[lev]
                _phase1_linear(xyz, idx8, sub8, f3, b, res, offs)
            else:
                _phase1_hashed(xyz, idx8, sub8, f3, b, lev)
            cps = [pltpu.async_copy(emb.at[idx8.at[b, c]], vals8.at[b, c],
                                    sems[b]) for c in range(8)]
            if pending is not None:
                pb, pcps, plev = pending
                for cp in pcps:
                    cp.wait()
                _phase2_accumulate(f3, sub8, vals8, outc, pb, 2 * plev)
            pending = (b, cps, lev)
        pb, pcps, plev = pending
        for cp in pcps:
            cp.wait()
        _phase2_accumulate(f3, sub8, vals8, outc, pb, 2 * plev)
        pltpu.sync_copy(outc, out.at[pl.ds(base * 2 * _NLEV, _CH * 2 * _NLEV)])
        return 0

    lax.fori_loop(0, _NCHUNK, chunk_body, 0, unroll=False)


def kernel(inputs, embeddings):
    inp_t = inputs.T
    embv = embeddings.reshape(-1, 128, 2).transpose(0, 2, 1).reshape(-1)
    embq = _relayout(embv).reshape(-1, _PAD)
    out = _hash_encode(jnp.asarray(inp_t), embq)
    return out.reshape(_B, 2 * _NLEV)


# speedup vs baseline: 5.6168x; 1.1105x over previous
import functools

import numpy as np
import jax
import jax.numpy as jnp
from jax import lax
from jax.experimental import pallas as pl
from jax.experimental.pallas import tpu as pltpu
from jax.experimental.pallas import tpu_sc as plsc

_B = 524288
_NW = 32
_PTS = _B // _NW
_CH = 512
_NCHUNK = _PTS // _CH
_PAD = 8
_L = 16
_NVEC = _CH // _L
_MASK = (1 << 19) - 1
_P1 = int(np.uint32(2654435761).view(np.int32))
_P2 = int(np.uint32(805459861).view(np.int32))
_HOFF0 = 299008
_NLEV = 16
_TOTAL_PARAMS = 7114752
_LIN = ((16, 0), (32, 4096), (64, 36864))

_mesh = plsc.VectorSubcoreMesh(core_axis_name="c", subcore_axis_name="s")
_cparams = pltpu.CompilerParams(
    needs_layout_passes=False, use_tc_tiling_on_sc=False
)

_TOTF = _TOTAL_PARAMS * 2
_RPW = _TOTF // _NW
_RNB = 193
_RITER = _RPW // (256 * _RNB)


@functools.partial(
    pl.kernel,
    out_type=jax.ShapeDtypeStruct((_TOTF,), jnp.float32),
    mesh=_mesh,
    scratch_types=[
        pltpu.VMEM((_RNB * 256,), jnp.float32),
        pltpu.VMEM((_RNB * 256,), jnp.float32),
        pltpu.SemaphoreType.DMA,
    ],
    compiler_params=_cparams,
)
def _relayout(embv, out, inbuf, outbuf, sem):
    wid = lax.axis_index("s") * 2 + lax.axis_index("c")
    base = wid * _RPW
    iota = lax.iota(jnp.int32, _L)
    pat = ((iota & 1) << 7) + (iota >> 1)

    def body(t, _):
        off = base + t * (_RNB * 256)
        pltpu.sync_copy(embv.at[pl.ds(off, _RNB * 256)], inbuf)

        def blk(b, _b):
            pb = pat + b * 256

            for g in range(16):
                outbuf[pl.ds(b * 256 + 16 * g, _L)] = plsc.load_gather(
                    inbuf, [pb + 8 * g])
            return 0

        lax.fori_loop(0, _RNB, blk, 0, unroll=False)
        pltpu.sync_copy(outbuf, out.at[pl.ds(off, _RNB * 256)])
        return 0

    lax.fori_loop(0, _RITER, body, 0, unroll=False)


def _phase1_hashed(xyz, idx8, sub8, f3, b, lev):
    resm1 = (16 << lev) - 1
    scale = float(resm1)
    offs = _HOFF0 + ((lev - 3) << 19)

    def body(i, _):
        s = i * _L
        x = xyz[0, pl.ds(s, _L)]
        y = xyz[1, pl.ds(s, _L)]
        z = xyz[2, pl.ds(s, _L)]
        px, py, pz = x * scale, y * scale, z * scale
        cx0 = px.astype(jnp.int32)
        cy0 = py.astype(jnp.int32)
        cz0 = pz.astype(jnp.int32)
        f3[b, pl.ds(s, _L)] = px - cx0.astype(jnp.float32)
        f3[b, pl.ds(_CH + s, _L)] = py - cy0.astype(jnp.float32)
        f3[b, pl.ds(2 * _CH + s, _L)] = pz - cz0.astype(jnp.float32)
        cx1 = jnp.minimum(cx0 + 1, resm1)
        cy1 = jnp.minimum(cy0 + 1, resm1)
        cz1 = jnp.minimum(cz0 + 1, resm1)
        hy0, hy1 = cy0 * _P1, cy1 * _P1
        hz0, hz1 = cz0 * _P2, cz1 * _P2
        hxy = ((cx0 ^ hy0, cx1 ^ hy0), (cx0 ^ hy1, cx1 ^ hy1))
        hz = (hz0, hz1)
        for c in range(8):
            bx, by, bz = c & 1, (c >> 1) & 1, c >> 2
            idx = ((hxy[by][bx] ^ hz[bz]) & _MASK) + offs
            idx8[b, c, pl.ds(s, _L)] = idx >> 2
            sub8[b, c, pl.ds(s, _L)] = (idx & 3) << 1
        return 0

    lax.fori_loop(0, _NVEC, body, 0, unroll=False)


def _phase1_linear(xyz, idx8, sub8, f3, b, res, offs):
    scale = float(res - 1)
    resm1 = res - 1
    s1, s2 = res, res * res

    def body(i, _):
        s = i * _L
        x = xyz[0, pl.ds(s, _L)]
        y = xyz[1, pl.ds(s, _L)]
        z = xyz[2, pl.ds(s, _L)]
        px, py, pz = x * scale, y * scale, z * scale
        cx0 = px.astype(jnp.int32)
        cy0 = py.astype(jnp.int32)
        cz0 = pz.astype(jnp.int32)
        f3[b, pl.ds(s, _L)] = px - cx0.astype(jnp.float32)
        f3[b, pl.ds(_CH + s, _L)] = py - cy0.astype(jnp.float32)
        f3[b, pl.ds(2 * _CH + s, _L)] = pz - cz0.astype(jnp.float32)
        cx1 = jnp.minimum(cx0 + 1, resm1)
        cy1 = jnp.minimum(cy0 + 1, resm1)
        cz1 = jnp.minimum(cz0 + 1, resm1)
        sxy = ((cx0 + cy0 * s1, cx1 + cy0 * s1), (cx0 + cy1 * s1, cx1 + cy1 * s1))
        sz = (cz0 * s2 + offs, cz1 * s2 + offs)
        for c in range(8):
            bx, by, bz = c & 1, (c >> 1) & 1, c >> 2
            idx = sxy[by][bx] + sz[bz]
            idx8[b, c, pl.ds(s, _L)] = idx >> 2
            sub8[b, c, pl.ds(s, _L)] = (idx & 3) << 1
        return 0

    lax.fori_loop(0, _NVEC, body, 0, unroll=False)


def _phase2_l0(f3, sub8, idx8, l0tab, outc, b):
    iota = lax.iota(jnp.int32, _L)

    def body(i, _):
        s = i * _L
        p = iota + s
        fx = f3[b, pl.ds(s, _L)]
        fy = f3[b, pl.ds(_CH + s, _L)]
        fz = f3[b, pl.ds(2 * _CH + s, _L)]
        gx, gy, gz = 1.0 - fx, 1.0 - fy, 1.0 - fz
        v = []
        for c in range(8):
            q = idx8[b, c, pl.ds(s, _L)]
            sub = sub8[b, c, pl.ds(s, _L)]
            v.append((plsc.load_gather(l0tab, [q, sub]),
                      plsc.load_gather(l0tab, [q, sub + 1])))
        pcol = p * (2 * _NLEV)
        for ch in (0, 1):
            u00 = v[0][ch] * gx + v[1][ch] * fx
            u10 = v[2][ch] * gx + v[3][ch] * fx
            u01 = v[4][ch] * gx + v[5][ch] * fx
            u11 = v[6][ch] * gx + v[7][ch] * fx
            m0 = u00 * gy + u10 * fy
            m1 = u01 * gy + u11 * fy
            plsc.store_scatter(outc, [pcol + ch], m0 * gz + m1 * fz)
        return 0

    lax.fori_loop(0, _NVEC, body, 0, unroll=False)


def _phase2_accumulate(f3, sub8, vals8, outc, b, lev2):
    iota = lax.iota(jnp.int32, _L)

    def body(i, _):
        s = i * _L
        p = iota + s
        fx = f3[b, pl.ds(s, _L)]
        fy = f3[b, pl.ds(_CH + s, _L)]
        fz = f3[b, pl.ds(2 * _CH + s, _L)]
        gx, gy, gz = 1.0 - fx, 1.0 - fy, 1.0 - fz
        subs = [sub8[b, c, pl.ds(s, _L)] for c in range(8)]
        v = [(plsc.load_gather(vals8.at[b, c], [p, subs[c]]),
              plsc.load_gather(vals8.at[b, c], [p, subs[c] + 1]))
             for c in range(8)]
        pcol = p * (2 * _NLEV) + lev2
        for ch in (0, 1):
            u00 = v[0][ch] * gx + v[1][ch] * fx
            u10 = v[2][ch] * gx + v[3][ch] * fx
            u01 = v[4][ch] * gx + v[5][ch] * fx
            u11 = v[6][ch] * gx + v[7][ch] * fx
            m0 = u00 * gy + u10 * fy
            m1 = u01 * gy + u11 * fy
            plsc.store_scatter(outc, [pcol + ch], m0 * gz + m1 * fz)
        return 0

    lax.fori_loop(0, _NVEC, body, 0, unroll=False)


@functools.partial(
    pl.kernel,
    out_type=jax.ShapeDtypeStruct((_B * 2 * _NLEV,), jnp.float32),
    mesh=_mesh,
    scratch_types=[
        pltpu.VMEM((3, _CH), jnp.float32),
        pltpu.VMEM((2, 8, _CH), jnp.int32),
        pltpu.VMEM((2, 8, _CH), jnp.int32),
        pltpu.VMEM((2, 3 * _CH), jnp.float32),
        pltpu.VMEM((2, 8, _CH, _PAD), jnp.float32),
        pltpu.VMEM((_CH * 2 * _NLEV,), jnp.float32),
        pltpu.VMEM((1024, _PAD), jnp.float32),
        pltpu.SemaphoreType.DMA,
        pltpu.SemaphoreType.DMA,
    ],
    compiler_params=_cparams,
)
def _hash_encode(inp_t, emb, out, xyz, idx8, sub8, f3, vals8, outc, l0tab,
                 sem0, sem1):
    wid = lax.axis_index("s") * 2 + lax.axis_index("c")
    sems = (sem0, sem1)
    pltpu.sync_copy(emb.at[pl.ds(0, 1024), :], l0tab)

    def chunk_body(k, _):
        base = wid * _PTS + k * _CH
        pltpu.sync_copy(inp_t.at[:, pl.ds(base, _CH)], xyz)
        pending = None
        for lev in range(_NLEV):
            b = lev & 1
            if lev < 3:
                res, offs = _LIN[lev]
                _phase1_linear(xyz, idx8, sub8, f3, b, res, offs)
            else:
                _phase1_hashed(xyz, idx8, sub8, f3, b, lev)
            if lev == 0:
                cps = []
            else:
                cps = [pltpu.async_copy(emb.at[idx8.at[b, c]], vals8.at[b, c],
                                        sems[b]) for c in range(8)]
            if pending is not None:
                pb, pcps, plev = pending
                for cp in pcps:
                    cp.wait()
                if plev == 0:
                    _phase2_l0(f3, sub8, idx8, l0tab, outc, pb)
                else:
                    _phase2_accumulate(f3, sub8, vals8, outc, pb, 2 * plev)
            pending = (b, cps, lev)
        pb, pcps, plev = pending
        for cp in pcps:
            cp.wait()
        _phase2_accumulate(f3, sub8, vals8, outc, pb, 2 * plev)
        pltpu.sync_copy(outc, out.at[pl.ds(base * 2 * _NLEV, _CH * 2 * _NLEV)])
        return 0

    lax.fori_loop(0, _NCHUNK, chunk_body, 0, unroll=False)


def kernel(inputs, embeddings):
    inp_t = inputs.T
    embv = embeddings.reshape(-1, 128, 2).transpose(0, 2, 1).reshape(-1)
    embq = _relayout(embv).reshape(-1, _PAD)
    out = _hash_encode(jnp.asarray(inp_t), embq)
    return out.reshape(_B, 2 * _NLEV)


# speedup vs baseline: 5.7579x; 1.0251x over previous
import functools

import numpy as np
import jax
import jax.numpy as jnp
from jax import lax
from jax.experimental import pallas as pl
from jax.experimental.pallas import tpu as pltpu
from jax.experimental.pallas import tpu_sc as plsc

_B = 524288
_NW = 32
_PTS = _B // _NW
_CH = 256
_NCHUNK = _PTS // _CH
_PAD = 8
_L = 16
_NVEC = _CH // _L
_MASK = (1 << 19) - 1
_P1 = int(np.uint32(2654435761).view(np.int32))
_P2 = int(np.uint32(805459861).view(np.int32))
_HOFF0 = 299008
_NLEV = 16
_TOTAL_PARAMS = 7114752
_LIN = ((16, 0), (32, 4096), (64, 36864))

_mesh = plsc.VectorSubcoreMesh(core_axis_name="c", subcore_axis_name="s")
_cparams = pltpu.CompilerParams(
    needs_layout_passes=False, use_tc_tiling_on_sc=False
)

_TOTF = _TOTAL_PARAMS * 2
_RPW = _TOTF // _NW
_RNB = 193
_RITER = _RPW // (256 * _RNB)


@functools.partial(
    pl.kernel,
    out_type=jax.ShapeDtypeStruct((_TOTF,), jnp.float32),
    mesh=_mesh,
    scratch_types=[
        pltpu.VMEM((_RNB * 256,), jnp.float32),
        pltpu.VMEM((_RNB * 256,), jnp.float32),
        pltpu.SemaphoreType.DMA,
    ],
    compiler_params=_cparams,
)
def _relayout(embv, out, inbuf, outbuf, sem):
    wid = lax.axis_index("s") * 2 + lax.axis_index("c")
    base = wid * _RPW
    iota = lax.iota(jnp.int32, _L)
    pat = ((iota & 1) << 7) + (iota >> 1)

    def body(t, _):
        off = base + t * (_RNB * 256)
        pltpu.sync_copy(embv.at[pl.ds(off, _RNB * 256)], inbuf)

        def blk(b, _b):
            pb = pat + b * 256

            for g in range(16):
                outbuf[pl.ds(b * 256 + 16 * g, _L)] = plsc.load_gather(
                    inbuf, [pb + 8 * g])
            return 0

        lax.fori_loop(0, _RNB, blk, 0, unroll=False)
        pltpu.sync_copy(outbuf, out.at[pl.ds(off, _RNB * 256)])
        return 0

    lax.fori_loop(0, _RITER, body, 0, unroll=False)


def _phase1_hashed(xyz, idx8, sub8, f3, b, lev):
    resm1 = (16 << lev) - 1
    scale = float(resm1)
    offs = _HOFF0 + ((lev - 3) << 19)

    def body(i, _):
        s = i * _L
        x = xyz[0, pl.ds(s, _L)]
        y = xyz[1, pl.ds(s, _L)]
        z = xyz[2, pl.ds(s, _L)]
        px, py, pz = x * scale, y * scale, z * scale
        cx0 = px.astype(jnp.int32)
        cy0 = py.astype(jnp.int32)
        cz0 = pz.astype(jnp.int32)
        f3[b, pl.ds(s, _L)] = px - cx0.astype(jnp.float32)
        f3[b, pl.ds(_CH + s, _L)] = py - cy0.astype(jnp.float32)
        f3[b, pl.ds(2 * _CH + s, _L)] = pz - cz0.astype(jnp.float32)
        cx1 = jnp.minimum(cx0 + 1, resm1)
        cy1 = jnp.minimum(cy0 + 1, resm1)
        cz1 = jnp.minimum(cz0 + 1, resm1)
        hy0, hy1 = cy0 * _P1, cy1 * _P1
        hz0, hz1 = cz0 * _P2, cz1 * _P2
        hxy = ((cx0 ^ hy0, cx1 ^ hy0), (cx0 ^ hy1, cx1 ^ hy1))
        hz = (hz0, hz1)
        for c in range(8):
            bx, by, bz = c & 1, (c >> 1) & 1, c >> 2
            idx = ((hxy[by][bx] ^ hz[bz]) & _MASK) + offs
            idx8[b, c, pl.ds(s, _L)] = idx >> 2
            sub8[b, c, pl.ds(s, _L)] = (idx & 3) << 1
        return 0

    lax.fori_loop(0, _NVEC, body, 0, unroll=False)


def _phase1_linear(xyz, idx8, sub8, f3, b, res, offs):
    scale = float(res - 1)
    resm1 = res - 1
    s1, s2 = res, res * res

    def body(i, _):
        s = i * _L
        x = xyz[0, pl.ds(s, _L)]
        y = xyz[1, pl.ds(s, _L)]
        z = xyz[2, pl.ds(s, _L)]
        px, py, pz = x * scale, y * scale, z * scale
        cx0 = px.astype(jnp.int32)
        cy0 = py.astype(jnp.int32)
        cz0 = pz.astype(jnp.int32)
        f3[b, pl.ds(s, _L)] = px - cx0.astype(jnp.float32)
        f3[b, pl.ds(_CH + s, _L)] = py - cy0.astype(jnp.float32)
        f3[b, pl.ds(2 * _CH + s, _L)] = pz - cz0.astype(jnp.float32)
        cx1 = jnp.minimum(cx0 + 1, resm1)
        cy1 = jnp.minimum(cy0 + 1, resm1)
        cz1 = jnp.minimum(cz0 + 1, resm1)
        sxy = ((cx0 + cy0 * s1, cx1 + cy0 * s1), (cx0 + cy1 * s1, cx1 + cy1 * s1))
        sz = (cz0 * s2 + offs, cz1 * s2 + offs)
        for c in range(8):
            bx, by, bz = c & 1, (c >> 1) & 1, c >> 2
            idx = sxy[by][bx] + sz[bz]
            idx8[b, c, pl.ds(s, _L)] = idx >> 2
            sub8[b, c, pl.ds(s, _L)] = (idx & 3) << 1
        return 0

    lax.fori_loop(0, _NVEC, body, 0, unroll=False)


def _phase2_staged(f3, sub8, idx8, tab, outc, b, lev2, qoff):
    iota = lax.iota(jnp.int32, _L)

    def body(i, _):
        s = i * _L
        p = iota + s
        fx = f3[b, pl.ds(s, _L)]
        fy = f3[b, pl.ds(_CH + s, _L)]
        fz = f3[b, pl.ds(2 * _CH + s, _L)]
        gx, gy, gz = 1.0 - fx, 1.0 - fy, 1.0 - fz
        v = []
        for c in range(8):
            q = idx8[b, c, pl.ds(s, _L)] - qoff
            sub = sub8[b, c, pl.ds(s, _L)]
            v.append((plsc.load_gather(tab, [q, sub]),
                      plsc.load_gather(tab, [q, sub + 1])))
        pcol = p * (2 * _NLEV) + lev2
        for ch in (0, 1):
            u00 = v[0][ch] * gx + v[1][ch] * fx
            u10 = v[2][ch] * gx + v[3][ch] * fx
            u01 = v[4][ch] * gx + v[5][ch] * fx
            u11 = v[6][ch] * gx + v[7][ch] * fx
            m0 = u00 * gy + u10 * fy
            m1 = u01 * gy + u11 * fy
            plsc.store_scatter(outc, [pcol + ch], m0 * gz + m1 * fz)
        return 0

    lax.fori_loop(0, _NVEC, body, 0, unroll=False)


def _phase2_accumulate(f3, sub8, vals8, outc, b, lev2):
    iota = lax.iota(jnp.int32, _L)

    def body(i, _):
        s = i * _L
        p = iota + s
        fx = f3[b, pl.ds(s, _L)]
        fy = f3[b, pl.ds(_CH + s, _L)]
        fz = f3[b, pl.ds(2 * _CH + s, _L)]
        gx, gy, gz = 1.0 - fx, 1.0 - fy, 1.0 - fz
        subs = [sub8[b, c, pl.ds(s, _L)] for c in range(8)]
        v = [(plsc.load_gather(vals8.at[b, c], [p, subs[c]]),
              plsc.load_gather(vals8.at[b, c], [p, subs[c] + 1]))
             for c in range(8)]
        pcol = p * (2 * _NLEV) + lev2
        for ch in (0, 1):
            u00 = v[0][ch] * gx + v[1][ch] * fx
            u10 = v[2][ch] * gx + v[3][ch] * fx
            u01 = v[4][ch] * gx + v[5][ch] * fx
            u11 = v[6][ch] * gx + v[7][ch] * fx
            m0 = u00 * gy + u10 * fy
            m1 = u01 * gy + u11 * fy
            plsc.store_scatter(outc, [pcol + ch], m0 * gz + m1 * fz)
        return 0

    lax.fori_loop(0, _NVEC, body, 0, unroll=False)


@functools.partial(
    pl.kernel,
    out_type=jax.ShapeDtypeStruct((_B * 2 * _NLEV,), jnp.float32),
    mesh=_mesh,
    scratch_types=[
        pltpu.VMEM((3, _CH), jnp.float32),
        pltpu.VMEM((2, 8, _CH), jnp.int32),
        pltpu.VMEM((2, 8, _CH), jnp.int32),
        pltpu.VMEM((2, 3 * _CH), jnp.float32),
        pltpu.VMEM((2, 8, _CH, _PAD), jnp.float32),
        pltpu.VMEM((_CH * 2 * _NLEV,), jnp.float32),
        pltpu.VMEM((1024, _PAD), jnp.float32),
        pltpu.VMEM((8192, _PAD), jnp.float32),
        pltpu.SemaphoreType.DMA,
        pltpu.SemaphoreType.DMA,
    ],
    compiler_params=_cparams,
)
def _hash_encode(inp_t, emb, out, xyz, idx8, sub8, f3, vals8, outc, l0tab,
                 l1tab, sem0, sem1):
    wid = lax.axis_index("s") * 2 + lax.axis_index("c")
    sems = (sem0, sem1)
    pltpu.sync_copy(emb.at[pl.ds(0, 1024), :], l0tab)
    pltpu.sync_copy(emb.at[pl.ds(1024, 8192), :], l1tab)

    def chunk_body(k, _):
        base = wid * _PTS + k * _CH
        pltpu.sync_copy(inp_t.at[:, pl.ds(base, _CH)], xyz)
        pending = None
        for lev in range(_NLEV):
            b = lev & 1
            if lev < 3:
                res, offs = _LIN[lev]
                _phase1_linear(xyz, idx8, sub8, f3, b, res, offs)
            else:
                _phase1_hashed(xyz, idx8, sub8, f3, b, lev)
            if lev <= 1:
                cps = []
            else:
                cps = [pltpu.async_copy(emb.at[idx8.at[b, c]], vals8.at[b, c],
                                        sems[b]) for c in range(8)]
            if pending is not None:
                pb, pcps, plev = pending
                for cp in pcps:
                    cp.wait()
                if plev == 0:
                    _phase2_staged(f3, sub8, idx8, l0tab, outc, pb, 0, 0)
                elif plev == 1:
                    _phase2_staged(f3, sub8, idx8, l1tab, outc, pb, 2, 1024)
                else:
                    _phase2_accumulate(f3, sub8, vals8, outc, pb, 2 * plev)
            pending = (b, cps, lev)
        pb, pcps, plev = pending
        for cp in pcps:
            cp.wait()
        _phase2_accumulate(f3, sub8, vals8, outc, pb, 2 * plev)
        pltpu.sync_copy(outc, out.at[pl.ds(base * 2 * _NLEV, _CH * 2 * _NLEV)])
        return 0

    lax.fori_loop(0, _NCHUNK, chunk_body, 0, unroll=False)


def kernel(inputs, embeddings):
    inp_t = inputs.T
    embv = embeddings.reshape(-1, 128, 2).transpose(0, 2, 1).reshape(-1)
    embq = _relayout(embv).reshape(-1, _PAD)
    out = _hash_encode(jnp.asarray(inp_t), embq)
    return out.reshape(_B, 2 * _NLEV)


# speedup vs baseline: 6.5299x; 1.1341x over previous
import functools

import numpy as np
import jax
import jax.numpy as jnp
from jax import lax
from jax.experimental import pallas as pl
from jax.experimental.pallas import tpu as pltpu
from jax.experimental.pallas import tpu_sc as plsc

_B = 524288
_NW = 32
_PTS = _B // _NW
_CH = 256
_NCHUNK = _PTS // _CH
_PAD = 8
_L = 16
_NVEC = _CH // _L
_MASK = (1 << 19) - 1
_P1 = int(np.uint32(2654435761).view(np.int32))
_P2 = int(np.uint32(805459861).view(np.int32))
_HOFF0 = 299008
_NLEV = 16
_TOTAL_PARAMS = 7114752
_LIN = ((16, 0), (32, 4096), (64, 36864))

_mesh = plsc.VectorSubcoreMesh(core_axis_name="c", subcore_axis_name="s")
_cparams = pltpu.CompilerParams(
    needs_layout_passes=False, use_tc_tiling_on_sc=False
)

_TOTF = _TOTAL_PARAMS * 2
_RPW = _TOTF // _NW
_RNB = 193
_RITER = _RPW // (256 * _RNB)


@functools.partial(
    pl.kernel,
    out_type=jax.ShapeDtypeStruct((_TOTF,), jnp.float32),
    mesh=_mesh,
    scratch_types=[
        pltpu.VMEM((_RNB * 256,), jnp.float32),
        pltpu.VMEM((_RNB * 256,), jnp.float32),
        pltpu.SemaphoreType.DMA,
    ],
    compiler_params=_cparams,
)
def _relayout(embv, out, inbuf, outbuf, sem):
    wid = lax.axis_index("s") * 2 + lax.axis_index("c")
    base = wid * _RPW
    iota = lax.iota(jnp.int32, _L)
    pat = ((iota & 1) << 7) + (iota >> 1)

    def body(t, _):
        off = base + t * (_RNB * 256)
        pltpu.sync_copy(embv.at[pl.ds(off, _RNB * 256)], inbuf)

        def blk(b, _b):
            pb = pat + b * 256

            for g in range(16):
                outbuf[pl.ds(b * 256 + 16 * g, _L)] = plsc.load_gather(
                    inbuf, [pb + 8 * g])
            return 0

        lax.fori_loop(0, _RNB, blk, 0, unroll=False)
        pltpu.sync_copy(outbuf, out.at[pl.ds(off, _RNB * 256)])
        return 0

    lax.fori_loop(0, _RITER, body, 0, unroll=False)


def _phase1_hashed(xyz, idx8, sub8, f3, b, lev):
    resm1 = (16 << lev) - 1
    scale = float(resm1)
    offs = _HOFF0 + ((lev - 3) << 19)

    def body(i, _):
        s = i * _L
        x = xyz[0, pl.ds(s, _L)]
        y = xyz[1, pl.ds(s, _L)]
        z = xyz[2, pl.ds(s, _L)]
        px, py, pz = x * scale, y * scale, z * scale
        cx0 = px.astype(jnp.int32)
        cy0 = py.astype(jnp.int32)
        cz0 = pz.astype(jnp.int32)
        f3[b, pl.ds(s, _L)] = px - cx0.astype(jnp.float32)
        f3[b, pl.ds(_CH + s, _L)] = py - cy0.astype(jnp.float32)
        f3[b, pl.ds(2 * _CH + s, _L)] = pz - cz0.astype(jnp.float32)
        cx1 = jnp.minimum(cx0 + 1, resm1)
        cy1 = jnp.minimum(cy0 + 1, resm1)
        cz1 = jnp.minimum(cz0 + 1, resm1)
        hy0, hy1 = cy0 * _P1, cy1 * _P1
        hz0, hz1 = cz0 * _P2, cz1 * _P2
        hxy = ((cx0 ^ hy0, cx1 ^ hy0), (cx0 ^ hy1, cx1 ^ hy1))
        hz = (hz0, hz1)
        for c in range(8):
            bx, by, bz = c & 1, (c >> 1) & 1, c >> 2
            idx = ((hxy[by][bx] ^ hz[bz]) & _MASK) + offs
            idx8[b, c, pl.ds(s, _L)] = idx >> 2
            sub8[b, c, pl.ds(s, _L)] = (idx & 3) << 1
        return 0

    lax.fori_loop(0, _NVEC, body, 0, unroll=False)


def _phase1_linear(xyz, idx8, sub8, f3, b, res, offs):
    scale = float(res - 1)
    resm1 = res - 1
    s1, s2 = res, res * res

    def body(i, _):
        s = i * _L
        x = xyz[0, pl.ds(s, _L)]
        y = xyz[1, pl.ds(s, _L)]
        z = xyz[2, pl.ds(s, _L)]
        px, py, pz = x * scale, y * scale, z * scale
        cx0 = px.astype(jnp.int32)
        cy0 = py.astype(jnp.int32)
        cz0 = pz.astype(jnp.int32)
        f3[b, pl.ds(s, _L)] = px - cx0.astype(jnp.float32)
        f3[b, pl.ds(_CH + s, _L)] = py - cy0.astype(jnp.float32)
        f3[b, pl.ds(2 * _CH + s, _L)] = pz - cz0.astype(jnp.float32)
        cx1 = jnp.minimum(cx0 + 1, resm1)
        cy1 = jnp.minimum(cy0 + 1, resm1)
        cz1 = jnp.minimum(cz0 + 1, resm1)
        sxy = ((cx0 + cy0 * s1, cx1 + cy0 * s1), (cx0 + cy1 * s1, cx1 + cy1 * s1))
        sz = (cz0 * s2 + offs, cz1 * s2 + offs)
        for c in range(8):
            bx, by, bz = c & 1, (c >> 1) & 1, c >> 2
            idx = sxy[by][bx] + sz[bz]
            idx8[b, c, pl.ds(s, _L)] = idx >> 2
            sub8[b, c, pl.ds(s, _L)] = (idx & 3) << 1
        return 0

    lax.fori_loop(0, _NVEC, body, 0, unroll=False)


def _phase2_staged(f3, sub8, idx8, tab, outc, b, lev2, qoff):
    iota = lax.iota(jnp.int32, _L)

    def body(i, _):
        s = i * _L
        p = iota + s
        fx = f3[b, pl.ds(s, _L)]
        fy = f3[b, pl.ds(_CH + s, _L)]
        fz = f3[b, pl.ds(2 * _CH + s, _L)]
        gx, gy, gz = 1.0 - fx, 1.0 - fy, 1.0 - fz
        v = []
        for c in range(8):
            q = idx8[b, c, pl.ds(s, _L)] - qoff
            sub = sub8[b, c, pl.ds(s, _L)]
            v.append((plsc.load_gather(tab, [q, sub]),
                      plsc.load_gather(tab, [q, sub + 1])))
        for ch in (0, 1):
            u00 = v[0][ch] * gx + v[1][ch] * fx
            u10 = v[2][ch] * gx + v[3][ch] * fx
            u01 = v[4][ch] * gx + v[5][ch] * fx
            u11 = v[6][ch] * gx + v[7][ch] * fx
            m0 = u00 * gy + u10 * fy
            m1 = u01 * gy + u11 * fy
            gch = lev2 + ch
            off = (i >> 3) * 1024 + (gch & 7) * 128 + ((i * _L) & 127)
            outc[gch >> 3, pl.ds(off, _L)] = m0 * gz + m1 * fz
        return 0

    lax.fori_loop(0, _NVEC, body, 0, unroll=False)


def _phase2_accumulate(f3, sub8, vals8, outc, b, lev2):
    iota = lax.iota(jnp.int32, _L)

    def body(i, _):
        s = i * _L
        p = iota + s
        fx = f3[b, pl.ds(s, _L)]
        fy = f3[b, pl.ds(_CH + s, _L)]
        fz = f3[b, pl.ds(2 * _CH + s, _L)]
        gx, gy, gz = 1.0 - fx, 1.0 - fy, 1.0 - fz
        subs = [sub8[b, c, pl.ds(s, _L)] for c in range(8)]
        v = [(plsc.load_gather(vals8.at[b, c], [p, subs[c]]),
              plsc.load_gather(vals8.at[b, c], [p, subs[c] + 1]))
             for c in range(8)]
        for ch in (0, 1):
            u00 = v[0][ch] * gx + v[1][ch] * fx
            u10 = v[2][ch] * gx + v[3][ch] * fx
            u01 = v[4][ch] * gx + v[5][ch] * fx
            u11 = v[6][ch] * gx + v[7][ch] * fx
            m0 = u00 * gy + u10 * fy
            m1 = u01 * gy + u11 * fy
            gch = lev2 + ch
            off = (i >> 3) * 1024 + (gch & 7) * 128 + ((i * _L) & 127)
            outc[gch >> 3, pl.ds(off, _L)] = m0 * gz + m1 * fz
        return 0

    lax.fori_loop(0, _NVEC, body, 0, unroll=False)


@functools.partial(
    pl.kernel,
    out_type=jax.ShapeDtypeStruct((_B * 2 * _NLEV,), jnp.float32),
    mesh=_mesh,
    scratch_types=[
        pltpu.VMEM((3, _CH), jnp.float32),
        pltpu.VMEM((2, 8, _CH), jnp.int32),
        pltpu.VMEM((2, 8, _CH), jnp.int32),
        pltpu.VMEM((2, 3 * _CH), jnp.float32),
        pltpu.VMEM((2, 8, _CH, _PAD), jnp.float32),
        pltpu.VMEM((4, 2 * 1024), jnp.float32),
        pltpu.VMEM((1024, _PAD), jnp.float32),
        pltpu.VMEM((8192, _PAD), jnp.float32),
        pltpu.SemaphoreType.DMA,
        pltpu.SemaphoreType.DMA,
    ],
    compiler_params=_cparams,
)
def _hash_encode(inp_t, emb, out, xyz, idx8, sub8, f3, vals8, outc, l0tab,
                 l1tab, sem0, sem1):
    wid = lax.axis_index("s") * 2 + lax.axis_index("c")
    sems = (sem0, sem1)
    pltpu.sync_copy(emb.at[pl.ds(0, 1024), :], l0tab)
    pltpu.sync_copy(emb.at[pl.ds(1024, 8192), :], l1tab)

    def chunk_body(k, _):
        base = wid * _PTS + k * _CH
        pltpu.sync_copy(inp_t.at[:, pl.ds(base, _CH)], xyz)
        pending = None
        for lev in range(_NLEV):
            b = lev & 1
            if lev < 3:
                res, offs = _LIN[lev]
                _phase1_linear(xyz, idx8, sub8, f3, b, res, offs)
            else:
                _phase1_hashed(xyz, idx8, sub8, f3, b, lev)
            if lev <= 1:
                cps = []
            else:
                cps = [pltpu.async_copy(emb.at[idx8.at[b, c]], vals8.at[b, c],
                                        sems[b]) for c in range(8)]
            if pending is not None:
                pb, pcps, plev = pending
                for cp in pcps:
                    cp.wait()
                if plev == 0:
                    _phase2_staged(f3, sub8, idx8, l0tab, outc, pb, 0, 0)
                elif plev == 1:
                    _phase2_staged(f3, sub8, idx8, l1tab, outc, pb, 2, 1024)
                else:
                    _phase2_accumulate(f3, sub8, vals8, outc, pb, 2 * plev)
            pending = (b, cps, lev)
        pb, pcps, plev = pending
        for cp in pcps:
            cp.wait()
        _phase2_accumulate(f3, sub8, vals8, outc, pb, 2 * plev)
        bq = base >> 7
        for a in range(4):
            pltpu.sync_copy(outc.at[a],
                            out.at[pl.ds((a * 4096 + bq) * 1024, 2048)])
        return 0

    lax.fori_loop(0, _NCHUNK, chunk_body, 0, unroll=False)


def kernel(inputs, embeddings):
    inp_t = inputs.T
    embv = embeddings.reshape(-1, 128, 2).transpose(0, 2, 1).reshape(-1)
    embq = _relayout(embv).reshape(-1, _PAD)
    out = _hash_encode(jnp.asarray(inp_t), embq)
    return (out.reshape(4, 4096, 8, 128)
               .transpose(1, 3, 0, 2)
               .reshape(_B, 2 * _NLEV))
